# baseline (speedup 1.0000x reference)
import jax
import jax.numpy as jnp
from jax.experimental import pallas as pl
from jax.experimental.pallas import tpu as pltpu

N = 4096
C = 32
BLOCK_ROWS = 128
NUM_BLOCKS = N // BLOCK_ROWS

LAP_SCALE = 2048.0
RHS_SCALE = 64.0


def _snn_body(x_ref, ld_ref, lu_ref, w_ref, out_ref, ldq, luq, p_buf, q_buf,
              yacc_buf):
    t = pl.program_id(0)
    i = pl.program_id(1)
    rows = pl.ds(i * BLOCK_ROWS, BLOCK_ROWS)
    w = w_ref[...]
    xb = x_ref[...].astype(jnp.bfloat16)

    @pl.when(t == 0)
    def _pass_down():
        ldb = ld_ref[...]
        u1 = jnp.dot(ldb.astype(jnp.bfloat16), xb,
                     preferred_element_type=jnp.float32)
        yacc_buf[rows, :] = (
            jnp.dot(x_ref[rows, :], w[:, :, 0], preferred_element_type=jnp.float32)
            + jnp.dot(u1, w[:, :, 1], preferred_element_type=jnp.float32)
        )
        p_buf[rows, :] = jnp.dot(u1, w[:, :, 2], preferred_element_type=jnp.float32)
        ldq[rows, :] = jnp.clip(jnp.round(ldb * LAP_SCALE),
                                -127.0, 127.0).astype(jnp.int8)

    @pl.when(t == 1)
    def _pass_up():
        lub = lu_ref[...]
        v1 = jnp.dot(lub.astype(jnp.bfloat16), xb,
                     preferred_element_type=jnp.float32)
        pq = jnp.clip(jnp.round(p_buf[...] * RHS_SCALE),
                      -127.0, 127.0).astype(jnp.int8)
        u2w = jnp.dot(ldq[rows, :], pq,
                      preferred_element_type=jnp.int32).astype(jnp.float32)
        yacc_buf[rows, :] += (
            jnp.dot(v1, w[:, :, 3], preferred_element_type=jnp.float32)
            + u2w * (1.0 / (LAP_SCALE * RHS_SCALE))
        )
        q_buf[rows, :] = jnp.dot(v1, w[:, :, 4], preferred_element_type=jnp.float32)
        luq[rows, :] = jnp.clip(jnp.round(lub * LAP_SCALE),
                                -127.0, 127.0).astype(jnp.int8)

    @pl.when(t == 2)
    def _tail():
        qq = jnp.clip(jnp.round(q_buf[...] * RHS_SCALE),
                      -127.0, 127.0).astype(jnp.int8)
        v2w = jnp.dot(luq[rows, :], qq,
                      preferred_element_type=jnp.int32).astype(jnp.float32)
        out_ref[...] = jax.nn.sigmoid(
            yacc_buf[rows, :] + v2w * (1.0 / (LAP_SCALE * RHS_SCALE)))


@jax.jit
def kernel(x, laplacian_down, laplacian_up, weight):
    last = NUM_BLOCKS - 1
    return pl.pallas_call(
        _snn_body,
        grid=(3, NUM_BLOCKS),
        in_specs=[
            pl.BlockSpec((N, C), lambda t, i: (0, 0)),
            pl.BlockSpec((BLOCK_ROWS, N),
                         lambda t, i: (jnp.where(t == 0, i, last), 0)),
            pl.BlockSpec((BLOCK_ROWS, N),
                         lambda t, i: (jnp.where(t == 0, 0,
                                                 jnp.where(t == 1, i, last)), 0)),
            pl.BlockSpec((C, C, 5), lambda t, i: (0, 0, 0)),
        ],
        out_specs=pl.BlockSpec((BLOCK_ROWS, C), lambda t, i: (i, 0)),
        out_shape=jax.ShapeDtypeStruct((N, C), jnp.float32),
        scratch_shapes=[
            pltpu.VMEM((N, N), jnp.int8),
            pltpu.VMEM((N, N), jnp.int8),
            pltpu.VMEM((N, C), jnp.float32),
            pltpu.VMEM((N, C), jnp.float32),
            pltpu.VMEM((N, C), jnp.float32),
        ],
        compiler_params=pltpu.CompilerParams(
            dimension_semantics=("arbitrary", "arbitrary"),
        ),
    )(x, laplacian_down, laplacian_up, weight)


# speedup vs baseline: 1.2195x; 1.2195x over previous
import jax
import jax.numpy as jnp
from jax.experimental import pallas as pl
from jax.experimental.pallas import tpu as pltpu

N = 4096
C = 32
BLOCK_ROWS = 512
NUM_BLOCKS = N // BLOCK_ROWS


def _pass1_body(x_ref, ld_ref, lu_ref, w_ref, p_ref, q_ref, yacc_ref):
    i = pl.program_id(0)
    rows = pl.ds(i * BLOCK_ROWS, BLOCK_ROWS)
    w = w_ref[...]
    xb = x_ref[...].astype(jnp.bfloat16)
    u1 = jnp.dot(ld_ref[...].astype(jnp.bfloat16), xb,
                 preferred_element_type=jnp.float32)
    v1 = jnp.dot(lu_ref[...].astype(jnp.bfloat16), xb,
                 preferred_element_type=jnp.float32)
    p_ref[...] = jnp.dot(u1, w[:, :, 2], preferred_element_type=jnp.float32)
    q_ref[...] = jnp.dot(v1, w[:, :, 4], preferred_element_type=jnp.float32)
    yacc_ref[...] = (
        jnp.dot(x_ref[rows, :], w[:, :, 0], preferred_element_type=jnp.float32)
        + jnp.dot(u1, w[:, :, 1], preferred_element_type=jnp.float32)
        + jnp.dot(v1, w[:, :, 3], preferred_element_type=jnp.float32)
    )


def _pass2_body(ld_ref, lu_ref, p_ref, q_ref, yacc_ref, out_ref):
    y = (
        yacc_ref[...]
        + jnp.dot(ld_ref[...].astype(jnp.bfloat16),
                  p_ref[...].astype(jnp.bfloat16),
                  preferred_element_type=jnp.float32)
        + jnp.dot(lu_ref[...].astype(jnp.bfloat16),
                  q_ref[...].astype(jnp.bfloat16),
                  preferred_element_type=jnp.float32)
    )
    out_ref[...] = jax.nn.sigmoid(y)


@jax.jit
def kernel(x, laplacian_down, laplacian_up, weight):
    small = jax.ShapeDtypeStruct((N, C), jnp.float32)
    p, q, yacc = pl.pallas_call(
        _pass1_body,
        grid=(NUM_BLOCKS,),
        in_specs=[
            pl.BlockSpec((N, C), lambda i: (0, 0)),
            pl.BlockSpec((BLOCK_ROWS, N), lambda i: (i, 0)),
            pl.BlockSpec((BLOCK_ROWS, N), lambda i: (i, 0)),
            pl.BlockSpec((C, C, 5), lambda i: (0, 0, 0)),
        ],
        out_specs=[
            pl.BlockSpec((BLOCK_ROWS, C), lambda i: (i, 0)),
            pl.BlockSpec((BLOCK_ROWS, C), lambda i: (i, 0)),
            pl.BlockSpec((BLOCK_ROWS, C), lambda i: (i, 0)),
        ],
        out_shape=[small, small, small],
        compiler_params=pltpu.CompilerParams(
            dimension_semantics=("parallel",),
        ),
    )(x, laplacian_down, laplacian_up, weight)
    return pl.pallas_call(
        _pass2_body,
        grid=(NUM_BLOCKS,),
        in_specs=[
            pl.BlockSpec((BLOCK_ROWS, N), lambda i: (i, 0)),
            pl.BlockSpec((BLOCK_ROWS, N), lambda i: (i, 0)),
            pl.BlockSpec((N, C), lambda i: (0, 0)),
            pl.BlockSpec((N, C), lambda i: (0, 0)),
            pl.BlockSpec((BLOCK_ROWS, C), lambda i: (i, 0)),
        ],
        out_specs=pl.BlockSpec((BLOCK_ROWS, C), lambda i: (i, 0)),
        out_shape=jax.ShapeDtypeStruct((N, C), jnp.float32),
        compiler_params=pltpu.CompilerParams(
            dimension_semantics=("parallel",),
        ),
    )(laplacian_down, laplacian_up, p, q, yacc)


# speedup vs baseline: 1.5464x; 1.2681x over previous
import jax
import jax.numpy as jnp
from jax.experimental import pallas as pl
from jax.experimental.pallas import tpu as pltpu

N = 4096
C = 32
BLOCK_ROWS = 256
NUM_BLOCKS = N // BLOCK_ROWS

F8 = jnp.float8_e4m3fn
LAP_SCALE = 256.0
INV_LAP_SCALE = 1.0 / LAP_SCALE


def _snn_body(x_ref, ld_ref, lu_ref, w_ref, out_ref, ldq, luq, u1b, v1b, zd):
    t = pl.program_id(0)
    i = pl.program_id(1)
    rows = pl.ds(i * BLOCK_ROWS, BLOCK_ROWS)
    xb = x_ref[...].astype(jnp.bfloat16)

    @pl.when(t == 0)
    def _pass_down():
        ldb = ld_ref[...]
        u1 = jnp.dot(ldb.astype(jnp.bfloat16), xb,
                     preferred_element_type=jnp.float32)
        u1b[rows, :] = u1.astype(jnp.bfloat16)
        ldq[rows, :] = (ldb * LAP_SCALE).astype(F8)

    @pl.when(t == 1)
    def _pass_up():
        lub = lu_ref[...]
        v1 = jnp.dot(lub.astype(jnp.bfloat16), xb,
                     preferred_element_type=jnp.float32)
        v1b[rows, :] = v1.astype(jnp.bfloat16)
        luq[rows, :] = (lub * LAP_SCALE).astype(F8)
        zd[rows, :] = jnp.dot(ldq[rows, :].astype(jnp.bfloat16), u1b[...],
                              preferred_element_type=jnp.float32)

    @pl.when(t == 2)
    def _tail():
        w = w_ref[...]
        zu = jnp.dot(luq[rows, :].astype(jnp.bfloat16), v1b[...],
                     preferred_element_type=jnp.float32)
        y = (
            jnp.dot(x_ref[rows, :], w[0:C, :], preferred_element_type=jnp.float32)
            + jnp.dot(u1b[rows, :], w[C:2 * C, :], preferred_element_type=jnp.float32)
            + jnp.dot(zd[rows, :] * INV_LAP_SCALE, w[2 * C:3 * C, :],
                      preferred_element_type=jnp.float32)
            + jnp.dot(v1b[rows, :], w[3 * C:4 * C, :], preferred_element_type=jnp.float32)
            + jnp.dot(zu * INV_LAP_SCALE, w[4 * C:5 * C, :],
                      preferred_element_type=jnp.float32)
        )
        out_ref[...] = jax.nn.sigmoid(y)


@jax.jit
def kernel(x, laplacian_down, laplacian_up, weight):
    last = NUM_BLOCKS - 1
    wt = jnp.transpose(weight, (2, 0, 1)).reshape(5 * C, C)
    return pl.pallas_call(
        _snn_body,
        grid=(3, NUM_BLOCKS),
        in_specs=[
            pl.BlockSpec((N, C), lambda t, i: (0, 0)),
            pl.BlockSpec((BLOCK_ROWS, N),
                         lambda t, i: (jnp.where(t == 0, i, last), 0)),
            pl.BlockSpec((BLOCK_ROWS, N),
                         lambda t, i: (jnp.where(t == 0, 0,
                                                 jnp.where(t == 1, i, last)), 0)),
            pl.BlockSpec((5 * C, C), lambda t, i: (0, 0)),
        ],
        out_specs=pl.BlockSpec((BLOCK_ROWS, C), lambda t, i: (i, 0)),
        out_shape=jax.ShapeDtypeStruct((N, C), jnp.float32),
        scratch_shapes=[
            pltpu.VMEM((N, N), F8),
            pltpu.VMEM((N, N), F8),
            pltpu.VMEM((N, C), jnp.bfloat16),
            pltpu.VMEM((N, C), jnp.bfloat16),
            pltpu.VMEM((N, C), jnp.float32),
        ],
        compiler_params=pltpu.CompilerParams(
            dimension_semantics=("arbitrary", "arbitrary"),
        ),
    )(x, laplacian_down, laplacian_up, wt)


# speedup vs baseline: 1.7389x; 1.1245x over previous
import jax
import jax.numpy as jnp
from jax.experimental import pallas as pl
from jax.experimental.pallas import tpu as pltpu

N = 4096
C = 32
BLOCK_ROWS = 256
NUM_BLOCKS = N // BLOCK_ROWS

F8 = jnp.float8_e4m3fn
LAP_SCALE = 256.0
INV_LAP_SCALE = 1.0 / LAP_SCALE


def _snn_body(x_ref, ld_ref, lu_ref, w_ref, out_ref, ldq, luq, u1b, v1b, zd):
    t = pl.program_id(0)
    i = pl.program_id(1)
    rows = pl.ds(i * BLOCK_ROWS, BLOCK_ROWS)
    DEF = jax.lax.Precision.DEFAULT

    @pl.when(t == 0)
    def _pass_down():
        ldb = ld_ref[...]
        u1 = jnp.dot(ldb, x_ref[...], precision=DEF,
                     preferred_element_type=jnp.float32)
        u1b[rows, :] = u1.astype(jnp.bfloat16)
        ldq[rows, :] = (ldb * LAP_SCALE).astype(F8)

    @pl.when(t == 1)
    def _pass_up():
        lub = lu_ref[...]
        v1 = jnp.dot(lub, x_ref[...], precision=DEF,
                     preferred_element_type=jnp.float32)
        v1b[rows, :] = v1.astype(jnp.bfloat16)
        luq[rows, :] = (lub * LAP_SCALE).astype(F8)
        zd[rows, :] = jnp.dot(ldq[rows, :], u1b[...].astype(F8),
                              preferred_element_type=jnp.float32)

    @pl.when(t == 2)
    def _tail():
        w = w_ref[...]
        zu = jnp.dot(luq[rows, :], v1b[...].astype(F8),
                     preferred_element_type=jnp.float32)
        y = (
            jnp.dot(x_ref[rows, :], w[0:C, :], preferred_element_type=jnp.float32)
            + jnp.dot(u1b[rows, :], w[C:2 * C, :], preferred_element_type=jnp.float32)
            + jnp.dot(zd[rows, :] * INV_LAP_SCALE, w[2 * C:3 * C, :],
                      preferred_element_type=jnp.float32)
            + jnp.dot(v1b[rows, :], w[3 * C:4 * C, :], preferred_element_type=jnp.float32)
            + jnp.dot(zu * INV_LAP_SCALE, w[4 * C:5 * C, :],
                      preferred_element_type=jnp.float32)
        )
        out_ref[...] = jax.nn.sigmoid(y)


@jax.jit
def kernel(x, laplacian_down, laplacian_up, weight):
    last = NUM_BLOCKS - 1
    wt = jnp.transpose(weight, (2, 0, 1)).reshape(5 * C, C)
    return pl.pallas_call(
        _snn_body,
        grid=(3, NUM_BLOCKS),
        in_specs=[
            pl.BlockSpec((N, C), lambda t, i: (0, 0)),
            pl.BlockSpec((BLOCK_ROWS, N),
                         lambda t, i: (jnp.where(t == 0, i, last), 0)),
            pl.BlockSpec((BLOCK_ROWS, N),
                         lambda t, i: (jnp.where(t == 0, 0,
                                                 jnp.where(t == 1, i, last)), 0)),
            pl.BlockSpec((5 * C, C), lambda t, i: (0, 0)),
        ],
        out_specs=pl.BlockSpec((BLOCK_ROWS, C), lambda t, i: (i, 0)),
        out_shape=jax.ShapeDtypeStruct((N, C), jnp.float32),
        scratch_shapes=[
            pltpu.VMEM((N, N), F8),
            pltpu.VMEM((N, N), F8),
            pltpu.VMEM((N, C), jnp.bfloat16),
            pltpu.VMEM((N, C), jnp.bfloat16),
            pltpu.VMEM((N, C), jnp.float32),
        ],
        compiler_params=pltpu.CompilerParams(
            dimension_semantics=("arbitrary", "arbitrary"),
        ),
    )(x, laplacian_down, laplacian_up, wt)


# speedup vs baseline: 1.9206x; 1.1045x over previous
import jax
import jax.numpy as jnp
from jax.experimental import pallas as pl
from jax.experimental.pallas import tpu as pltpu

N = 4096
C = 32
BLOCK_ROWS = 512
NUM_BLOCKS = N // BLOCK_ROWS

F8 = jnp.float8_e4m3fn
LAP_SCALE = 256.0
INV_LAP_SCALE = 1.0 / LAP_SCALE
DEF = jax.lax.Precision.DEFAULT


def _snn_body(x_ref, ld_ref, lu_ref, w_ref, out_ref,
              buf0, buf1, ldq, luq, u1b, v1b, u1q, v1q, zd,
              sem0, sem1):
    bufs = (buf0, buf1)
    sems = (sem0, sem1)
    mats = [ld_ref] * NUM_BLOCKS + [lu_ref] * NUM_BLOCKS
    n_steps = 2 * NUM_BLOCKS

    def fetch(s):
        blk = s % NUM_BLOCKS
        cp = pltpu.make_async_copy(
            mats[s].at[pl.ds(blk * BLOCK_ROWS, BLOCK_ROWS), :],
            bufs[s % 2], sems[s % 2])
        cp.start()
        return cp

    pending = {0: fetch(0)}
    for s in range(n_steps):
        if s + 1 < n_steps:
            pending[s + 1] = fetch(s + 1)
        pending.pop(s).wait()
        blk = s % NUM_BLOCKS
        rows = pl.ds(blk * BLOCK_ROWS, BLOCK_ROWS)
        b = bufs[s % 2][...]
        r1 = jnp.dot(b, x_ref[...], precision=DEF,
                     preferred_element_type=jnp.float32)
        if s < NUM_BLOCKS:
            u1b[rows, :] = r1.astype(jnp.bfloat16)
            u1q[rows, :] = r1.astype(F8)
            ldq[rows, :] = (b * LAP_SCALE).astype(F8)
        else:
            v1b[rows, :] = r1.astype(jnp.bfloat16)
            v1q[rows, :] = r1.astype(F8)
            luq[rows, :] = (b * LAP_SCALE).astype(F8)
            zd[rows, :] = jnp.dot(ldq[rows, :], u1q[...],
                                  preferred_element_type=jnp.float32)

    w = w_ref[...]
    for j in range(NUM_BLOCKS):
        rows = pl.ds(j * BLOCK_ROWS, BLOCK_ROWS)
        zu = jnp.dot(luq[rows, :], v1q[...],
                     preferred_element_type=jnp.float32)
        y = (
            jnp.dot(x_ref[rows, :], w[0:C, :], preferred_element_type=jnp.float32)
            + jnp.dot(u1b[rows, :], w[C:2 * C, :], preferred_element_type=jnp.float32)
            + jnp.dot(zd[rows, :] * INV_LAP_SCALE, w[2 * C:3 * C, :],
                      preferred_element_type=jnp.float32)
            + jnp.dot(v1b[rows, :], w[3 * C:4 * C, :], preferred_element_type=jnp.float32)
            + jnp.dot(zu * INV_LAP_SCALE, w[4 * C:5 * C, :],
                      preferred_element_type=jnp.float32)
        )
        out_ref[rows, :] = jax.nn.sigmoid(y)


@jax.jit
def kernel(x, laplacian_down, laplacian_up, weight):
    wt = jnp.transpose(weight, (2, 0, 1)).reshape(5 * C, C)
    return pl.pallas_call(
        _snn_body,
        grid=(1,),
        in_specs=[
            pl.BlockSpec((N, C), lambda s: (0, 0)),
            pl.BlockSpec(memory_space=pl.ANY),
            pl.BlockSpec(memory_space=pl.ANY),
            pl.BlockSpec((5 * C, C), lambda s: (0, 0)),
        ],
        out_specs=pl.BlockSpec((N, C), lambda s: (0, 0)),
        out_shape=jax.ShapeDtypeStruct((N, C), jnp.float32),
        scratch_shapes=[
            pltpu.VMEM((BLOCK_ROWS, N), jnp.float32),
            pltpu.VMEM((BLOCK_ROWS, N), jnp.float32),
            pltpu.VMEM((N, N), F8),
            pltpu.VMEM((N, N), F8),
            pltpu.VMEM((N, C), jnp.bfloat16),
            pltpu.VMEM((N, C), jnp.bfloat16),
            pltpu.VMEM((N, C), F8),
            pltpu.VMEM((N, C), F8),
            pltpu.VMEM((N, C), jnp.float32),
            pltpu.SemaphoreType.DMA,
            pltpu.SemaphoreType.DMA,
        ],
    )(x, laplacian_down, laplacian_up, wt)


# speedup vs baseline: 2.0457x; 1.0651x over previous
import jax
import jax.numpy as jnp
from jax.experimental import pallas as pl
from jax.experimental.pallas import tpu as pltpu

N = 4096
C = 32
BLOCK_ROWS = 256
NUM_BLOCKS = N // BLOCK_ROWS
NBUF = 4
TAIL_ROWS = 512
NUM_TAIL = N // TAIL_ROWS

F8 = jnp.float8_e4m3fn
LAP_SCALE = 256.0
DEF = jax.lax.Precision.DEFAULT


def _snn_body(x_ref, ld_ref, lu_ref, w_ref, out_ref,
              buf0, buf1, buf2, buf3, ldq, luq, u1f, v1f, u1q, v1q, zd,
              sem0, sem1, sem2, sem3):
    bufs = (buf0, buf1, buf2, buf3)
    sems = (sem0, sem1, sem2, sem3)
    mats = [ld_ref] * NUM_BLOCKS + [lu_ref] * NUM_BLOCKS
    n_steps = 2 * NUM_BLOCKS

    def fetch(s):
        blk = s % NUM_BLOCKS
        cp = pltpu.make_async_copy(
            mats[s].at[pl.ds(blk * BLOCK_ROWS, BLOCK_ROWS), :],
            bufs[s % NBUF], sems[s % NBUF])
        cp.start()
        return cp

    pending = {s: fetch(s) for s in range(NBUF - 1)}
    for s in range(n_steps):
        if s + NBUF - 1 < n_steps:
            pending[s + NBUF - 1] = fetch(s + NBUF - 1)
        pending.pop(s).wait()
        blk = s % NUM_BLOCKS
        rows = pl.ds(blk * BLOCK_ROWS, BLOCK_ROWS)
        b = bufs[s % NBUF][...]
        r1 = jnp.dot(b, x_ref[...], precision=DEF,
                     preferred_element_type=jnp.float32)
        if s < NUM_BLOCKS:
            u1f[rows, :] = r1
            u1q[rows, :] = r1.astype(F8)
            ldq[rows, :] = (b * LAP_SCALE).astype(F8)
        else:
            v1f[rows, :] = r1
            v1q[rows, :] = r1.astype(F8)
            luq[rows, :] = (b * LAP_SCALE).astype(F8)
            zd[rows, :] = jnp.dot(ldq[rows, :], u1q[...],
                                  preferred_element_type=jnp.float32)

    w = w_ref[...]
    for j in range(NUM_TAIL):
        rows = pl.ds(j * TAIL_ROWS, TAIL_ROWS)
        zu = jnp.dot(luq[rows, :], v1q[...],
                     preferred_element_type=jnp.float32)
        cat = jnp.concatenate(
            [x_ref[rows, :], u1f[rows, :], zd[rows, :], v1f[rows, :], zu],
            axis=1)
        y = jnp.dot(cat, w, precision=DEF, preferred_element_type=jnp.float32)
        out_ref[rows, :] = jax.nn.sigmoid(y)


@jax.jit
def kernel(x, laplacian_down, laplacian_up, weight):
    wt = jnp.transpose(weight, (2, 0, 1)).reshape(5 * C, C)
    scale = jnp.ones((5, 1, 1), jnp.float32).at[2].set(1.0 / LAP_SCALE)
    scale = scale.at[4].set(1.0 / LAP_SCALE)
    wt = (wt.reshape(5, C, C) * scale).reshape(5 * C, C)
    return pl.pallas_call(
        _snn_body,
        grid=(1,),
        in_specs=[
            pl.BlockSpec((N, C), lambda s: (0, 0)),
            pl.BlockSpec(memory_space=pl.ANY),
            pl.BlockSpec(memory_space=pl.ANY),
            pl.BlockSpec((5 * C, C), lambda s: (0, 0)),
        ],
        out_specs=pl.BlockSpec((N, C), lambda s: (0, 0)),
        out_shape=jax.ShapeDtypeStruct((N, C), jnp.float32),
        scratch_shapes=[
            pltpu.VMEM((BLOCK_ROWS, N), jnp.float32),
            pltpu.VMEM((BLOCK_ROWS, N), jnp.float32),
            pltpu.VMEM((BLOCK_ROWS, N), jnp.float32),
            pltpu.VMEM((BLOCK_ROWS, N), jnp.float32),
            pltpu.VMEM((N, N), F8),
            pltpu.VMEM((N, N), F8),
            pltpu.VMEM((N, C), jnp.float32),
            pltpu.VMEM((N, C), jnp.float32),
            pltpu.VMEM((N, C), F8),
            pltpu.VMEM((N, C), F8),
            pltpu.VMEM((N, C), jnp.float32),
            pltpu.SemaphoreType.DMA,
            pltpu.SemaphoreType.DMA,
            pltpu.SemaphoreType.DMA,
            pltpu.SemaphoreType.DMA,
        ],
    )(x, laplacian_down, laplacian_up, wt)
